# Initial kernel scaffold; baseline (speedup 1.0000x reference)
import jax
import jax.numpy as jnp
from jax.experimental import pallas as pl


def kernel(x, W, b, Wr, br, A, Bm):
    raise NotImplementedError("write your pallas kernel here")


# speedup vs baseline: 3.4541x; 3.4541x over previous
import functools

import jax
import jax.numpy as jnp
from jax.experimental import pallas as pl

D_MODEL = 768
OUT_DIM = 768
NUM_EXPERTS = 8
RANK = 8
SCALING = 16 / 8
QUESTION_START = 611
SEQ = 2048
N_QUESTION = (SEQ - 1) - QUESTION_START


def _moe_kernel(x_ref, w_ref, b_ref, wr_ref, br_ref, aall_ref, bmt_ref, out_ref):
    xb = x_ref[0]

    row = jax.lax.broadcasted_iota(jnp.int32, (SEQ, 1), 0)
    mask = ((row >= QUESTION_START) & (row < SEQ - 1)).astype(jnp.float32)
    xagg = jnp.sum(xb * mask, axis=0, keepdims=True) * (1.0 / N_QUESTION)

    logits = jax.lax.dot_general(
        xagg, wr_ref[...], (((1,), (1,)), ((), ())),
        preferred_element_type=jnp.float32) + br_ref[...]
    m = jnp.max(logits, axis=-1, keepdims=True)
    e = jnp.exp(logits - m)
    routing = e / jnp.sum(e, axis=-1, keepdims=True)

    rows = jax.lax.broadcasted_iota(jnp.int32, (NUM_EXPERTS, NUM_EXPERTS * RANK), 0)
    cols = jax.lax.broadcasted_iota(jnp.int32, (NUM_EXPERTS, NUM_EXPERTS * RANK), 1)
    sel = (cols // RANK == rows).astype(jnp.float32)
    w64 = jax.lax.dot_general(routing, sel, (((1,), (0,)), ((), ())),
                              preferred_element_type=jnp.float32)
    bw = bmt_ref[...] * (w64 * SCALING)

    meff = w_ref[...] + jax.lax.dot_general(
        bw, aall_ref[...], (((1,), (0,)), ((), ())),
        preferred_element_type=jnp.float32)

    out = jax.lax.dot_general(
        xb, meff, (((1,), (1,)), ((), ())),
        preferred_element_type=jnp.float32) + b_ref[...]
    out_ref[0] = out


@jax.jit
def kernel(x, W, b, Wr, br, A, Bm):
    B, S, D = x.shape
    aall = A.reshape(NUM_EXPERTS * RANK, D)
    bmt = jnp.transpose(Bm, (1, 0, 2)).reshape(OUT_DIM, NUM_EXPERTS * RANK)
    b2 = b.reshape(1, OUT_DIM)
    br2 = br.reshape(1, NUM_EXPERTS)

    grid = (B,)
    return pl.pallas_call(
        _moe_kernel,
        grid=grid,
        in_specs=[
            pl.BlockSpec((1, S, D), lambda i: (i, 0, 0)),
            pl.BlockSpec((OUT_DIM, D), lambda i: (0, 0)),
            pl.BlockSpec((1, OUT_DIM), lambda i: (0, 0)),
            pl.BlockSpec((NUM_EXPERTS, D), lambda i: (0, 0)),
            pl.BlockSpec((1, NUM_EXPERTS), lambda i: (0, 0)),
            pl.BlockSpec((NUM_EXPERTS * RANK, D), lambda i: (0, 0)),
            pl.BlockSpec((OUT_DIM, NUM_EXPERTS * RANK), lambda i: (0, 0)),
        ],
        out_specs=pl.BlockSpec((1, S, OUT_DIM), lambda i: (i, 0, 0)),
        out_shape=jax.ShapeDtypeStruct((B, S, OUT_DIM), jnp.float32),
    )(x, W, b2, Wr, br2, aall, bmt)


# speedup vs baseline: 3.4572x; 1.0009x over previous
import functools

import jax
import jax.numpy as jnp
from jax.experimental import pallas as pl
from jax.experimental.pallas import tpu as pltpu

D_MODEL = 768
OUT_DIM = 768
NUM_EXPERTS = 8
RANK = 8
SCALING = 16 / 8
QUESTION_START = 611
SEQ = 2048
N_QUESTION = (SEQ - 1) - QUESTION_START


def _moe_kernel(x_ref, w_ref, b_ref, wr_ref, br_ref, aall_ref, bmt_ref, out_ref):
    xb = x_ref[0]

    row = jax.lax.broadcasted_iota(jnp.int32, (SEQ, 1), 0)
    mask = ((row >= QUESTION_START) & (row < SEQ - 1)).astype(jnp.float32)
    xagg = jnp.sum(xb * mask, axis=0, keepdims=True) * (1.0 / N_QUESTION)

    logits = jax.lax.dot_general(
        xagg, wr_ref[...], (((1,), (1,)), ((), ())),
        preferred_element_type=jnp.float32) + br_ref[...]
    m = jnp.max(logits, axis=-1, keepdims=True)
    e = jnp.exp(logits - m)
    routing = e / jnp.sum(e, axis=-1, keepdims=True)

    rows = jax.lax.broadcasted_iota(jnp.int32, (NUM_EXPERTS, NUM_EXPERTS * RANK), 0)
    cols = jax.lax.broadcasted_iota(jnp.int32, (NUM_EXPERTS, NUM_EXPERTS * RANK), 1)
    sel = (cols // RANK == rows).astype(jnp.float32)
    w64 = jax.lax.dot_general(routing, sel, (((1,), (0,)), ((), ())),
                              preferred_element_type=jnp.float32)
    bw = bmt_ref[...] * (w64 * SCALING)

    meff = w_ref[...] + jax.lax.dot_general(
        bw, aall_ref[...], (((1,), (0,)), ((), ())),
        preferred_element_type=jnp.float32)

    out = jax.lax.dot_general(
        xb.astype(jnp.bfloat16), meff.astype(jnp.bfloat16),
        (((1,), (1,)), ((), ())),
        preferred_element_type=jnp.float32) + b_ref[...]
    out_ref[0] = out


@jax.jit
def kernel(x, W, b, Wr, br, A, Bm):
    B, S, D = x.shape
    aall = A.reshape(NUM_EXPERTS * RANK, D)
    bmt = jnp.transpose(Bm, (1, 0, 2)).reshape(OUT_DIM, NUM_EXPERTS * RANK)
    b2 = b.reshape(1, OUT_DIM)
    br2 = br.reshape(1, NUM_EXPERTS)

    grid = (B,)
    return pl.pallas_call(
        _moe_kernel,
        grid=grid,
        in_specs=[
            pl.BlockSpec((1, S, D), lambda i: (i, 0, 0)),
            pl.BlockSpec((OUT_DIM, D), lambda i: (0, 0)),
            pl.BlockSpec((1, OUT_DIM), lambda i: (0, 0)),
            pl.BlockSpec((NUM_EXPERTS, D), lambda i: (0, 0)),
            pl.BlockSpec((1, NUM_EXPERTS), lambda i: (0, 0)),
            pl.BlockSpec((NUM_EXPERTS * RANK, D), lambda i: (0, 0)),
            pl.BlockSpec((OUT_DIM, NUM_EXPERTS * RANK), lambda i: (0, 0)),
        ],
        out_specs=pl.BlockSpec((1, S, OUT_DIM), lambda i: (i, 0, 0)),
        out_shape=jax.ShapeDtypeStruct((B, S, OUT_DIM), jnp.float32),
        compiler_params=pltpu.CompilerParams(
            dimension_semantics=("parallel",)),
    )(x, W, b2, Wr, br2, aall, bmt)
